# Initial kernel scaffold; baseline (speedup 1.0000x reference)
import functools

import jax
import jax.numpy as jnp
from jax import lax
from jax.experimental import pallas as pl
from jax.experimental.pallas import tpu as pltpu
from jax.experimental.pallas import tpu_sc as plsc

N = 10000
E = 320000
D = 128
NC = 2
NS = 16
NW = NC * NS
EPT = E // NW
CH = 80
NCHUNK = EPT // CH
RPT = N // NS
RCH = 125
NRCH = RPT // RCH

_mesh = plsc.VectorSubcoreMesh(
    core_axis_name="c", subcore_axis_name="s", num_cores=NC, num_subcores=NS)


def _zero_fill(ref, nrows):
  zeros = jnp.zeros((16,), jnp.float32)
  ncol = ref.shape[1] // 16

  def body(i, _):
    for k in range(ncol):
      ref[i, pl.ds(k * 16, 16)] = zeros
    return 0

  lax.fori_loop(0, nrows, body, 0)


@functools.partial(
    pl.kernel,
    out_type=[
        jax.ShapeDtypeStruct((NW, NCHUNK, CH), jnp.float32),
        jax.ShapeDtypeStruct((2, N), jnp.float32),
    ],
    mesh=_mesh,
    scratch_types=[
        pltpu.VMEM((NCHUNK, CH), jnp.float32),
        pltpu.VMEM((NCHUNK, CH), jnp.int32),
        pltpu.VMEM((125, 16), jnp.float32),
        pltpu.VMEM_SHARED((N,), jnp.float32),
    ],
)
def _sc_deg(p_hbm, dst_hbm, pw_hbm, degp_hbm, pb, dstb, zb, acc):
  c = lax.axis_index("c")
  s = lax.axis_index("s")
  wid = c * NS + s

  @pl.when(s == 0)
  def _():
    _zero_fill(zb, 125)
    def zloop(j, _):
      pltpu.sync_copy(zb, acc.at[pl.ds(j * 2000, 2000)].reshape(125, 16))
      return 0
    lax.fori_loop(0, N // 2000, zloop, 0)

  pltpu.sync_copy(p_hbm.at[wid], pb)
  pltpu.sync_copy(dst_hbm.at[wid], dstb)

  def sig_body(i, _):
    for k in range(CH // 16):
      v = pb[i, pl.ds(k * 16, 16)]
      pb[i, pl.ds(k * 16, 16)] = 1.0 / (1.0 + jnp.exp(-v))
    return 0
  lax.fori_loop(0, NCHUNK, sig_body, 0)

  pltpu.sync_copy(pb, pw_hbm.at[wid])

  plsc.subcore_barrier()

  def sc_body(j, _):
    pltpu.sync_copy(pb.at[j], acc.at[dstb.at[j]], add=True)
    return 0
  lax.fori_loop(0, NCHUNK, sc_body, 0)

  plsc.subcore_barrier()

  @pl.when(s == 0)
  def _():
    pltpu.sync_copy(acc, degp_hbm.at[c])


@functools.partial(
    pl.kernel,
    out_type=jax.ShapeDtypeStruct((2, N, D), jnp.float32),
    mesh=_mesh,
    scratch_types=[
        pltpu.VMEM((NCHUNK, CH), jnp.int32),
        pltpu.VMEM((NCHUNK, CH), jnp.int32),
        pltpu.VMEM((NCHUNK, CH), jnp.float32),
        pltpu.VMEM((CH, D), jnp.float32),
        pltpu.VMEM((RCH, D), jnp.float32),
        pltpu.VMEM_SHARED((N, D), jnp.float32),
        pltpu.SemaphoreType.DMA,
    ],
)
def _sc_prop(hs_hbm, src_hbm, dst_hbm, pw_hbm, a_hbm,
             srcb, dstb, pwb, rows, zrows, acc, sem):
  c = lax.axis_index("c")
  s = lax.axis_index("s")
  wid = c * NS + s

  _zero_fill(zrows, RCH)
  def zloop(k, _):
    pltpu.sync_copy(zrows, acc.at[pl.ds(s * RPT + k * RCH, RCH)])
    return 0
  lax.fori_loop(0, NRCH, zloop, 0)

  pltpu.sync_copy(src_hbm.at[wid], srcb)
  pltpu.sync_copy(dst_hbm.at[wid], dstb)
  pltpu.sync_copy(pw_hbm.at[wid], pwb)

  plsc.subcore_barrier()

  def chunk_body(j, _):
    pltpu.async_copy(hs_hbm.at[srcb.at[j]], rows, sem).wait()

    def scale_body(e, _):
      sc = pwb[j, e]
      for k in range(D // 16):
        rows[e, pl.ds(k * 16, 16)] = rows[e, pl.ds(k * 16, 16)] * sc
      return 0
    lax.fori_loop(0, CH, scale_body, 0)

    pltpu.sync_copy(rows, acc.at[dstb.at[j]], add=True)
    return 0
  lax.fori_loop(0, NCHUNK, chunk_body, 0)

  plsc.subcore_barrier()

  def dloop(k, _):
    r0 = s * RPT + k * RCH
    pltpu.sync_copy(acc.at[pl.ds(r0, RCH)], a_hbm.at[c, pl.ds(r0, RCH)])
    return 0
  lax.fori_loop(0, NRCH, dloop, 0)


BM = 1000


def _tc_pre_body(x_ref, wt_ref, d0_ref, d1_ref, hs_ref, dis_ref):
  deg = d0_ref[...] + d1_ref[...] + 1.0
  dis = lax.rsqrt(deg)
  h = jnp.dot(x_ref[...], wt_ref[...], preferred_element_type=jnp.float32)
  hs_ref[...] = h * dis
  dis_ref[...] = dis


def _tc_mid_body(a_ref, hs_ref, dis_ref, b_ref, wt_ref, out_ref):
  dis = dis_ref[...]
  t = dis * (a_ref[0] + a_ref[1] + hs_ref[...]) + b_ref[...]
  t = jnp.maximum(t, 0.0)
  out_ref[...] = jnp.dot(t, wt_ref[...],
                         preferred_element_type=jnp.float32) * dis


def _tc_final_body(a_ref, hs_ref, dis_ref, b_ref, out_ref):
  dis = dis_ref[...]
  out_ref[...] = dis * (a_ref[0] + a_ref[1] + hs_ref[...]) + b_ref[...]


def _col_spec():
  return pl.BlockSpec((BM, 1), lambda i: (i, 0))


def _mat_spec():
  return pl.BlockSpec((BM, D), lambda i: (i, 0))


def _full_spec(shape):
  return pl.BlockSpec(shape, lambda i: tuple(0 for _ in shape))


_tc_pre = pl.pallas_call(
    _tc_pre_body,
    grid=(N // BM,),
    in_specs=[_mat_spec(), _full_spec((D, D)), _col_spec(), _col_spec()],
    out_specs=[_mat_spec(), _col_spec()],
    out_shape=[jax.ShapeDtypeStruct((N, D), jnp.float32),
               jax.ShapeDtypeStruct((N, 1), jnp.float32)],
)

_tc_mid = pl.pallas_call(
    _tc_mid_body,
    grid=(N // BM,),
    in_specs=[pl.BlockSpec((2, BM, D), lambda i: (0, i, 0)), _mat_spec(),
              _col_spec(), _full_spec((1, D)), _full_spec((D, D))],
    out_specs=_mat_spec(),
    out_shape=jax.ShapeDtypeStruct((N, D), jnp.float32),
)

_tc_final = pl.pallas_call(
    _tc_final_body,
    grid=(N // BM,),
    in_specs=[pl.BlockSpec((2, BM, D), lambda i: (0, i, 0)), _mat_spec(),
              _col_spec(), _full_spec((1, D))],
    out_specs=_mat_spec(),
    out_shape=jax.ShapeDtypeStruct((N, D), jnp.float32),
)


@jax.jit
def kernel(x, edge_index, P_vec, W1, b1, W2, b2):
  src = edge_index[0].astype(jnp.int32).reshape(NW, NCHUNK, CH)
  dst = edge_index[1].astype(jnp.int32).reshape(NW, NCHUNK, CH)
  p3 = P_vec.astype(jnp.float32).reshape(NW, NCHUNK, CH)

  pw3, degp = _sc_deg(p3, dst)
  d0 = degp[0][:, None]
  d1 = degp[1][:, None]

  hs1, dis = _tc_pre(x, W1.T, d0, d1)
  a1 = _sc_prop(hs1, src, dst, pw3)
  hs2 = _tc_mid(a1, hs1, dis, b1.reshape(1, D), W2.T)
  a2 = _sc_prop(hs2, src, dst, pw3)
  out = _tc_final(a2, hs2, dis, b2.reshape(1, D))
  return out


# speedup vs baseline: 9.7400x; 9.7400x over previous
import functools

import jax
import jax.numpy as jnp
from jax import lax
from jax.experimental import pallas as pl
from jax.experimental.pallas import tpu as pltpu
from jax.experimental.pallas import tpu_sc as plsc

N = 10000
E = 320000
D = 128
NC = 2
NS = 16
NW = NC * NS
EPT = E // NW
CH = 80
NCHUNK = EPT // CH
DTILES = 10
RPT = N // DTILES
RCH = 200
NRCH = RPT // RCH
EPS = E // NS
NCHP = EPS // CH
DH = D // 2
EBLK = 10
NBLK = NCHP // EBLK
EBLKD = 25
NBLKD = NCHUNK // EBLKD

_mesh = plsc.VectorSubcoreMesh(
    core_axis_name="c", subcore_axis_name="s", num_cores=NC, num_subcores=NS)


def _zero_fill(ref, nrows):
  zeros = jnp.zeros((16,), jnp.float32)
  ncol = ref.shape[1] // 16

  def body(i, _):
    for k in range(ncol):
      ref[i, pl.ds(k * 16, 16)] = zeros
    return 0

  lax.fori_loop(0, nrows, body, 0)


def _zero_fill_1d(ref, n):
  zeros = jnp.zeros((16,), jnp.float32)

  def body(i, _):
    ref[pl.ds(i * 16, 16)] = zeros
    return 0

  lax.fori_loop(0, n // 16, body, 0)


@functools.partial(
    pl.kernel,
    out_type=[
        jax.ShapeDtypeStruct((NW, NBLKD, EBLKD, CH), jnp.float32),
        jax.ShapeDtypeStruct((2, N), jnp.float32),
    ],
    mesh=_mesh,
    scratch_types=[
        pltpu.VMEM((EBLKD, CH), jnp.float32),
        pltpu.VMEM((EBLKD, CH), jnp.int32),
        pltpu.VMEM_SHARED((N,), jnp.float32),
    ],
)
def _sc_deg(p_hbm, dst_hbm, zn_hbm, pw_hbm, degp_hbm, pb, dstb, acc):
  c = lax.axis_index("c")
  s = lax.axis_index("s")
  wid = c * NS + s

  @pl.when(s == 0)
  def _():
    pltpu.sync_copy(zn_hbm, acc)

  plsc.subcore_barrier()

  def blk_body(b, _):
    pltpu.sync_copy(p_hbm.at[wid, b], pb)
    pltpu.sync_copy(dst_hbm.at[wid, b], dstb)

    def sig_body(i, _):
      for k in range(CH // 16):
        v = pb[i, pl.ds(k * 16, 16)]
        pb[i, pl.ds(k * 16, 16)] = 1.0 / (1.0 + jnp.exp(-v))
      return 0
    lax.fori_loop(0, EBLKD, sig_body, 0)

    pltpu.sync_copy(pb, pw_hbm.at[wid, b])

    def sc_body(j, _):
      pltpu.sync_copy(pb.at[j], acc.at[dstb.at[j]], add=True)
      return 0
    lax.fori_loop(0, EBLKD, sc_body, 0)
    return 0
  lax.fori_loop(0, NBLKD, blk_body, 0)

  plsc.subcore_barrier()

  @pl.when(s == 0)
  def _():
    pltpu.sync_copy(acc, degp_hbm.at[c])


@functools.partial(
    pl.kernel,
    out_type=jax.ShapeDtypeStruct((2, N, DH), jnp.float32),
    mesh=_mesh,
    scratch_types=[
        pltpu.VMEM((EBLK, CH), jnp.int32),
        pltpu.VMEM((EBLK, CH), jnp.int32),
        pltpu.VMEM((EBLK, CH), jnp.float32),
        pltpu.VMEM((CH, D), jnp.float32),
        pltpu.VMEM((CH, DH), jnp.float32),
        pltpu.VMEM_SHARED((N, DH), jnp.float32),
        pltpu.SemaphoreType.DMA,
    ],
)
def _sc_prop(hs_hbm, src_hbm, dst_hbm, pw_hbm, zh_hbm, a_hbm,
             srcb, dstb, pwb, rows, rowsh, acc, sem):
  c = lax.axis_index("c")
  s = lax.axis_index("s")

  @pl.when(s < DTILES)
  def _():
    r0 = s * RPT
    pltpu.sync_copy(zh_hbm.at[pl.ds(r0, RPT)], acc.at[pl.ds(r0, RPT)])

  plsc.subcore_barrier()

  def blk_body(b, _):
    pltpu.sync_copy(src_hbm.at[s, b], srcb)
    pltpu.sync_copy(dst_hbm.at[s, b], dstb)
    pltpu.sync_copy(pw_hbm.at[s, b], pwb)

    def chunk_body(j, _):
      pltpu.async_copy(hs_hbm.at[srcb.at[j]], rows, sem).wait()

      def make_scale(off):
        def scale_all():
          def scale_body(u, _):
            pv = pwb[j, pl.ds(u * 16, 16)]
            for l in range(16):
              sc = pv[l]
              e = u * 16 + l
              for k in range(DH // 16):
                rowsh[e, pl.ds(k * 16, 16)] = (
                    rows[e, pl.ds(off + k * 16, 16)] * sc)
            return 0
          lax.fori_loop(0, CH // 16, scale_body, 0)
        return scale_all
      pl.when(c == 0)(make_scale(0))
      pl.when(c == 1)(make_scale(DH))

      pltpu.sync_copy(rowsh, acc.at[dstb.at[j]], add=True)
      return 0
    lax.fori_loop(0, EBLK, chunk_body, 0)
    return 0
  lax.fori_loop(0, NBLK, blk_body, 0)

  plsc.subcore_barrier()

  @pl.when(s < DTILES)
  def _():
    r0 = s * RPT
    pltpu.sync_copy(acc.at[pl.ds(r0, RPT)], a_hbm.at[c, pl.ds(r0, RPT)])


BM = 1000


def _tc_pre_body(x_ref, wt_ref, d0_ref, d1_ref, hs_ref, dis_ref):
  deg = d0_ref[...] + d1_ref[...] + 1.0
  dis = lax.rsqrt(deg)
  h = jnp.dot(x_ref[...], wt_ref[...], preferred_element_type=jnp.float32)
  hs_ref[...] = h * dis
  dis_ref[...] = dis


def _tc_step_body(a_ref, hs_ref, dis_ref, b_ref, wt_ref, out_ref, hsn_ref):
  dis = dis_ref[...]
  a = jnp.concatenate([a_ref[0], a_ref[1]], axis=1)
  out = dis * (a + hs_ref[...]) + b_ref[...]
  out_ref[...] = out
  t = jnp.maximum(out, 0.0)
  hsn_ref[...] = jnp.dot(t, wt_ref[...],
                         preferred_element_type=jnp.float32) * dis


def _col_spec():
  return pl.BlockSpec((BM, 1), lambda i: (i, 0))


def _mat_spec():
  return pl.BlockSpec((BM, D), lambda i: (i, 0))


def _half_spec():
  return pl.BlockSpec((2, BM, DH), lambda i: (0, i, 0))


def _full_spec(shape):
  return pl.BlockSpec(shape, lambda i: tuple(0 for _ in shape))


_MAT_SHAPE = jax.ShapeDtypeStruct((N, D), jnp.float32)

_tc_pre = pl.pallas_call(
    _tc_pre_body,
    grid=(N // BM,),
    in_specs=[_mat_spec(), _full_spec((D, D)), _col_spec(), _col_spec()],
    out_specs=[_mat_spec(), _col_spec()],
    out_shape=[_MAT_SHAPE, jax.ShapeDtypeStruct((N, 1), jnp.float32)],
)

_tc_step = pl.pallas_call(
    _tc_step_body,
    grid=(N // BM,),
    in_specs=[_half_spec(), _mat_spec(), _col_spec(),
              _full_spec((1, D)), _full_spec((D, D))],
    out_specs=[_mat_spec(), _mat_spec()],
    out_shape=[_MAT_SHAPE, _MAT_SHAPE],
)


@jax.jit
def kernel(x, edge_index, P_vec, W1, b1, W2, b2):
  src32 = edge_index[0].astype(jnp.int32)
  dst32 = edge_index[1].astype(jnp.int32)
  dstw = dst32.reshape(NW, NBLKD, EBLKD, CH)
  p3 = P_vec.astype(jnp.float32).reshape(NW, NBLKD, EBLKD, CH)
  srcs = src32.reshape(NS, NBLK, EBLK, CH)
  dsts = dst32.reshape(NS, NBLK, EBLK, CH)

  zn = jnp.zeros((N,), jnp.float32)
  zh = jnp.zeros((N, DH), jnp.float32)
  pw3, degp = _sc_deg(p3, dstw, zn)
  pws = pw3.reshape(NS, NBLK, EBLK, CH)
  d0 = degp[0][:, None]
  d1 = degp[1][:, None]

  hs1, dis = _tc_pre(x, W1.T, d0, d1)

  b_stack = jnp.stack([b1.reshape(1, D), b2.reshape(1, D)])
  wt_stack = jnp.stack([W2.T, W2.T])

  def step(hs, xs):
    b_k, wt_k = xs
    a = _sc_prop(hs, srcs, dsts, pws, zh)
    out_k, hs_next = _tc_step(a, hs, dis, b_k, wt_k)
    return hs_next, out_k

  _, outs = lax.scan(step, hs1, (b_stack, wt_stack))
  return outs[1]
